# baseline (speedup 1.0000x reference)
import functools
import math

import jax
import jax.numpy as jnp
import numpy as np
from jax import lax
from jax.experimental import pallas as pl
from jax.experimental.pallas import tpu as pltpu
from jax.experimental.pallas import tpu_sc as plsc

HIDDEN = 256
ANS_NUM = 5000
OCR_NUM = 100
BATCH = 128
SEQ = 128
LN_EPS = 1e-12
LANES = 16

OCR_BASE = 5600
TAB_ROWS = OCR_BASE + BATCH * OCR_NUM
TAB_BLK = 800
OCR_BLK = 8
N_ANS_STEP = OCR_BASE // TAB_BLK
N_STEP = TAB_ROWS // TAB_BLK


def _make_pe(d_model=HIDDEN, max_len=SEQ):
    position = np.arange(max_len, dtype=np.float64)[:, None]
    div_term = np.exp(np.arange(0, d_model, 2, dtype=np.float64) * (-math.log(10000.0) / d_model))
    pe = np.zeros((max_len, d_model), dtype=np.float32)
    pe[:, 0::2] = np.sin(position / div_term)
    pe[:, 1::2] = np.cos(position / div_term)
    return pe


_PE2 = np.concatenate([_make_pe(), _make_pe()], axis=0)


def _ln(x, w, b):
    n = x.shape[-1]
    mu = jnp.mean(x, axis=-1, keepdims=True)
    var = jnp.maximum(jnp.mean(x * x, axis=-1, keepdims=True) - mu * mu, 0.0)
    return (x - mu) * lax.rsqrt(var + LN_EPS) * w + b


def _tab_body(ans_ref, ocr_ref, p_ref, o_ref):
    g = pl.program_id(0)

    @pl.when(g < N_ANS_STEP)
    def _():
        o_ref[...] = _ln(ans_ref[...], p_ref[0], p_ref[1])

    @pl.when(g >= N_ANS_STEP)
    def _():
        o_ref[...] = _ln(ocr_ref[...], p_ref[2], p_ref[3])


def _tab_call(ans_emb, ocr_flat, p):
    return pl.pallas_call(
        _tab_body,
        grid=(N_STEP,),
        in_specs=[
            pl.BlockSpec((TAB_BLK, HIDDEN), lambda g: (jnp.minimum(g, N_ANS_STEP - 1), 0)),
            pl.BlockSpec((TAB_BLK, HIDDEN),
                         lambda g: (jnp.clip(g - N_ANS_STEP, 0, BATCH * OCR_NUM // TAB_BLK - 1), 0)),
            pl.BlockSpec((4, HIDDEN), lambda g: (0, 0)),
        ],
        out_specs=pl.BlockSpec((TAB_BLK, HIDDEN), lambda g: (g, 0)),
        out_shape=jax.ShapeDtypeStruct((TAB_ROWS, HIDDEN), jnp.float32),
    )(ans_emb, ocr_flat, p)


def _emb_body(pe_ref, tt_ref, p_ref, o_ref):
    rid = lax.broadcasted_iota(jnp.int32, (2 * SEQ, 1), 0)
    x = pe_ref[...] + jnp.where(rid < SEQ, tt_ref[0], tt_ref[1])
    o_ref[...] = _ln(x, p_ref[0], p_ref[1])


def _emb_call(pe2, tt_table, p):
    return pl.pallas_call(
        _emb_body,
        grid=(1,),
        in_specs=[
            pl.BlockSpec((2 * SEQ, HIDDEN), lambda g: (0, 0)),
            pl.BlockSpec((2, HIDDEN), lambda g: (0, 0)),
            pl.BlockSpec((2, HIDDEN), lambda g: (0, 0)),
        ],
        out_specs=pl.BlockSpec((2 * SEQ, HIDDEN), lambda g: (0, 0)),
        out_shape=jax.ShapeDtypeStruct((2 * SEQ, HIDDEN), jnp.float32),
    )(pe2, tt_table, p)


def _sc_gather(table, emb_tab, labels):
    info = plsc.get_sparse_core_info()
    nc, ns = info.num_cores, info.num_subcores
    nw = nc * ns
    b_per_w = BATCH // nw
    mesh = plsc.VectorSubcoreMesh(core_axis_name="c", subcore_axis_name="s")

    chunk = 64
    n_chunk = b_per_w * SEQ // chunk

    @functools.partial(
        pl.kernel,
        mesh=mesh,
        out_type=jax.ShapeDtypeStruct((BATCH, SEQ, HIDDEN), jnp.float32),
        scratch_types=[
            pltpu.VMEM((b_per_w, SEQ), jnp.int32),
            pltpu.VMEM((2, chunk), jnp.int32),
            pltpu.VMEM((2, chunk), jnp.int32),
            pltpu.VMEM((2 * SEQ, HIDDEN), jnp.float32),
            pltpu.VMEM((2, chunk, HIDDEN), jnp.float32),
            pltpu.SemaphoreType.DMA,
            pltpu.SemaphoreType.DMA,
        ],
    )
    def k(table_hbm, emb_hbm, lbl_hbm, out_hbm, lbl_v, gi_v, ei_v, emb_v, rows_v, sem_g, sem_s):
        wid = lax.axis_index("s") * nc + lax.axis_index("c")
        emb_cp = pltpu.async_copy(emb_hbm, emb_v, sem_g)
        pltpu.sync_copy(lbl_hbm.at[pl.ds(wid * b_per_w, b_per_w)], lbl_v)

        def fire_gather(c):
            bi, half = c // 2, c % 2
            b = wid * b_per_w + bi
            slot = c % 2
            for j in range(chunk // LANES):
                v = lbl_v[bi, pl.ds(half * chunk + j * LANES, LANES)]
                is_ocr = v >= ANS_NUM
                gi_v[slot, pl.ds(j * LANES, LANES)] = jnp.where(
                    is_ocr, v + (OCR_BASE - ANS_NUM) + b * OCR_NUM, v)
                s_vec = lax.iota(jnp.int32, LANES) + (half * chunk + j * LANES)
                ei_v[slot, pl.ds(j * LANES, LANES)] = jnp.where(is_ocr, s_vec + SEQ, s_vec)
            return pltpu.async_copy(table_hbm.at[gi_v.at[slot]],
                                    rows_v.at[slot], sem_g)

        gathers = [None] * n_chunk
        scats = [None] * n_chunk
        gathers[0] = fire_gather(0)
        emb_cp.wait()
        for c in range(n_chunk):
            slot = c % 2
            if c + 1 < n_chunk:
                if c >= 1:
                    scats[c - 1].wait()
                gathers[c + 1] = fire_gather(c + 1)
            gathers[c].wait()

            def add_group(g, carry, slot=slot):
                ei16 = ei_v[slot, pl.ds(g * LANES, LANES)]
                for l in range(LANES):
                    s = g * LANES + l
                    eoff = ei16[l]
                    for h in range(HIDDEN // LANES):
                        sl = pl.ds(h * LANES, LANES)
                        rows_v[slot, s, sl] = rows_v[slot, s, sl] + emb_v[eoff, sl]
                return carry

            lax.fori_loop(0, chunk // LANES, add_group, 0)
            bi, half = c // 2, c % 2
            b = wid * b_per_w + bi
            scats[c] = pltpu.async_copy(rows_v.at[slot],
                                        out_hbm.at[b, pl.ds(half * chunk, chunk)], sem_s)
        scats[n_chunk - 2].wait()
        scats[n_chunk - 1].wait()

    return k(table, emb_tab, labels)


def kernel(ans_emb, ocr_emb, labels, tt_table, ans_ln_w, ans_ln_b, ocr_ln_w, ocr_ln_b, emb_ln_w, emb_ln_b):
    ocr_flat = ocr_emb.reshape(BATCH * OCR_NUM, HIDDEN)
    table = _tab_call(ans_emb, ocr_flat, jnp.stack([ans_ln_w, ans_ln_b, ocr_ln_w, ocr_ln_b]))
    emb_tab = _emb_call(jnp.asarray(_PE2), tt_table, jnp.stack([emb_ln_w, emb_ln_b]))
    return _sc_gather(table, emb_tab, labels.astype(jnp.int32))


# speedup vs baseline: 1.2162x; 1.2162x over previous
import functools
import math

import jax
import jax.numpy as jnp
import numpy as np
from jax import lax
from jax.experimental import pallas as pl
from jax.experimental.pallas import tpu as pltpu
from jax.experimental.pallas import tpu_sc as plsc

HIDDEN = 256
ANS_NUM = 5000
OCR_NUM = 100
BATCH = 128
SEQ = 128
LN_EPS = 1e-12
LANES = 16

OCR_BASE = 5600
TAB_ROWS = OCR_BASE + BATCH * OCR_NUM
TAB_BLK = 800
OCR_BLK = 8
N_ANS_STEP = OCR_BASE // TAB_BLK
N_STEP = TAB_ROWS // TAB_BLK


def _make_pe(d_model=HIDDEN, max_len=SEQ):
    position = np.arange(max_len, dtype=np.float64)[:, None]
    div_term = np.exp(np.arange(0, d_model, 2, dtype=np.float64) * (-math.log(10000.0) / d_model))
    pe = np.zeros((max_len, d_model), dtype=np.float32)
    pe[:, 0::2] = np.sin(position / div_term)
    pe[:, 1::2] = np.cos(position / div_term)
    return pe


_PE2 = np.concatenate([_make_pe(), _make_pe()], axis=0)


def _ln(x, w, b):
    n = x.shape[-1]
    mu = jnp.mean(x, axis=-1, keepdims=True)
    var = jnp.maximum(jnp.mean(x * x, axis=-1, keepdims=True) - mu * mu, 0.0)
    return (x - mu) * lax.rsqrt(var + LN_EPS) * w + b


def _tab_body(ans_ref, ocr_ref, p_ref, o_ref):
    g = pl.program_id(0)

    @pl.when(g < N_ANS_STEP)
    def _():
        o_ref[...] = _ln(ans_ref[...], p_ref[0], p_ref[1])

    @pl.when(g >= N_ANS_STEP)
    def _():
        o_ref[...] = _ln(ocr_ref[...], p_ref[2], p_ref[3])


def _tab_call(ans_emb, ocr_flat, p):
    return pl.pallas_call(
        _tab_body,
        grid=(N_STEP,),
        in_specs=[
            pl.BlockSpec((TAB_BLK, HIDDEN), lambda g: (jnp.minimum(g, N_ANS_STEP - 1), 0)),
            pl.BlockSpec((TAB_BLK, HIDDEN),
                         lambda g: (jnp.clip(g - N_ANS_STEP, 0, BATCH * OCR_NUM // TAB_BLK - 1), 0)),
            pl.BlockSpec((4, HIDDEN), lambda g: (0, 0)),
        ],
        out_specs=pl.BlockSpec((TAB_BLK, HIDDEN), lambda g: (g, 0)),
        out_shape=jax.ShapeDtypeStruct((TAB_ROWS, HIDDEN), jnp.float32),
    )(ans_emb, ocr_flat, p)


def _emb_body(pe_ref, tt_ref, p_ref, o_ref):
    rid = lax.broadcasted_iota(jnp.int32, (2 * SEQ, 1), 0)
    x = pe_ref[...] + jnp.where(rid < SEQ, tt_ref[0], tt_ref[1])
    o_ref[...] = _ln(x, p_ref[0], p_ref[1])


def _emb_call(pe2, tt_table, p):
    return pl.pallas_call(
        _emb_body,
        grid=(1,),
        in_specs=[
            pl.BlockSpec((2 * SEQ, HIDDEN), lambda g: (0, 0)),
            pl.BlockSpec((2, HIDDEN), lambda g: (0, 0)),
            pl.BlockSpec((2, HIDDEN), lambda g: (0, 0)),
        ],
        out_specs=pl.BlockSpec((2 * SEQ, HIDDEN), lambda g: (0, 0)),
        out_shape=jax.ShapeDtypeStruct((2 * SEQ, HIDDEN), jnp.float32),
    )(pe2, tt_table, p)


def _sc_gather(table, emb_tab, labels):
    info = plsc.get_sparse_core_info()
    nc, ns = info.num_cores, info.num_subcores
    nw = nc * ns
    b_per_w = BATCH // nw
    mesh = plsc.VectorSubcoreMesh(core_axis_name="c", subcore_axis_name="s")

    chunk = 64
    n_chunk = b_per_w * SEQ // chunk

    @functools.partial(
        pl.kernel,
        mesh=mesh,
        out_type=jax.ShapeDtypeStruct((BATCH, SEQ, HIDDEN), jnp.float32),
        scratch_types=[
            pltpu.VMEM((b_per_w, SEQ), jnp.int32),
            pltpu.VMEM((2, chunk), jnp.int32),
            pltpu.VMEM((2, chunk), jnp.int32),
            pltpu.VMEM((2, 2 * chunk, HIDDEN), jnp.float32),
            pltpu.SemaphoreType.DMA,
            pltpu.SemaphoreType.DMA,
        ],
    )
    def k(table_hbm, emb_hbm, lbl_hbm, out_hbm, lbl_v, gi_v, ei_v, rows_v, sem_g, sem_s):
        wid = lax.axis_index("s") * nc + lax.axis_index("c")
        pltpu.sync_copy(lbl_hbm.at[pl.ds(wid * b_per_w, b_per_w)], lbl_v)

        def fire_gather(c):
            bi, half = c // 2, c % 2
            b = wid * b_per_w + bi
            slot = c % 2
            for j in range(chunk // LANES):
                v = lbl_v[bi, pl.ds(half * chunk + j * LANES, LANES)]
                is_ocr = v >= ANS_NUM
                gi_v[slot, pl.ds(j * LANES, LANES)] = jnp.where(
                    is_ocr, v + (OCR_BASE - ANS_NUM) + b * OCR_NUM, v)
                s_vec = lax.iota(jnp.int32, LANES) + (half * chunk + j * LANES)
                ei_v[slot, pl.ds(j * LANES, LANES)] = jnp.where(is_ocr, s_vec + SEQ, s_vec)
            g1 = pltpu.async_copy(table_hbm.at[gi_v.at[slot]],
                                  rows_v.at[slot, pl.ds(0, chunk)], sem_g)
            g2 = pltpu.async_copy(emb_hbm.at[ei_v.at[slot]],
                                  rows_v.at[slot, pl.ds(chunk, chunk)], sem_g)
            return g1, g2

        gathers = [None] * n_chunk
        scats = [None] * n_chunk
        gathers[0] = fire_gather(0)
        for c in range(n_chunk):
            slot = c % 2
            if c + 1 < n_chunk:
                if c >= 1:
                    scats[c - 1].wait()
                gathers[c + 1] = fire_gather(c + 1)
            g1, g2 = gathers[c]
            g1.wait()
            g2.wait()

            def add_row(s, carry, slot=slot):
                for h in range(HIDDEN // LANES):
                    sl = pl.ds(h * LANES, LANES)
                    rows_v[slot, s, sl] = rows_v[slot, s, sl] + rows_v[slot, s + chunk, sl]
                return carry

            lax.fori_loop(0, chunk, add_row, 0)
            bi, half = c // 2, c % 2
            b = wid * b_per_w + bi
            scats[c] = pltpu.async_copy(rows_v.at[slot, pl.ds(0, chunk)],
                                        out_hbm.at[b, pl.ds(half * chunk, chunk)], sem_s)
        scats[n_chunk - 2].wait()
        scats[n_chunk - 1].wait()

    return k(table, emb_tab, labels)


def kernel(ans_emb, ocr_emb, labels, tt_table, ans_ln_w, ans_ln_b, ocr_ln_w, ocr_ln_b, emb_ln_w, emb_ln_b):
    ocr_flat = ocr_emb.reshape(BATCH * OCR_NUM, HIDDEN)
    table = _tab_call(ans_emb, ocr_flat, jnp.stack([ans_ln_w, ans_ln_b, ocr_ln_w, ocr_ln_b]))
    emb_tab = _emb_call(jnp.asarray(_PE2), tt_table, jnp.stack([emb_ln_w, emb_ln_b]))
    return _sc_gather(table, emb_tab, labels.astype(jnp.int32))


# speedup vs baseline: 1.3054x; 1.0733x over previous
import functools
import math

import jax
import jax.numpy as jnp
import numpy as np
from jax import lax
from jax.experimental import pallas as pl
from jax.experimental.pallas import tpu as pltpu
from jax.experimental.pallas import tpu_sc as plsc

HIDDEN = 256
ANS_NUM = 5000
OCR_NUM = 100
BATCH = 128
SEQ = 128
LN_EPS = 1e-12
LANES = 16

OCR_BASE = 5600
TAB_ROWS = OCR_BASE + BATCH * OCR_NUM
TAB_BLK = 800
OCR_BLK = 8
N_ANS_STEP = OCR_BASE // TAB_BLK
N_STEP = TAB_ROWS // TAB_BLK


def _make_pe(d_model=HIDDEN, max_len=SEQ):
    position = np.arange(max_len, dtype=np.float64)[:, None]
    div_term = np.exp(np.arange(0, d_model, 2, dtype=np.float64) * (-math.log(10000.0) / d_model))
    pe = np.zeros((max_len, d_model), dtype=np.float32)
    pe[:, 0::2] = np.sin(position / div_term)
    pe[:, 1::2] = np.cos(position / div_term)
    return pe


_PE2 = np.concatenate([_make_pe(), _make_pe()], axis=0)


def _ln(x, w, b):
    n = x.shape[-1]
    mu = jnp.mean(x, axis=-1, keepdims=True)
    var = jnp.maximum(jnp.mean(x * x, axis=-1, keepdims=True) - mu * mu, 0.0)
    return (x - mu) * lax.rsqrt(var + LN_EPS) * w + b


def _tab_body(ans_ref, ocr_ref, p_ref, o_ref):
    g = pl.program_id(0)

    @pl.when(g < N_ANS_STEP)
    def _():
        o_ref[...] = _ln(ans_ref[...], p_ref[0], p_ref[1])

    @pl.when(g >= N_ANS_STEP)
    def _():
        y = _ln(ocr_ref[...], p_ref[2], p_ref[3])
        for i in range(OCR_BLK):
            o_ref[pl.ds(i * OCR_NUM, OCR_NUM), :] = y[i]


def _tab_call(ans_emb, ocr_emb, p):
    return pl.pallas_call(
        _tab_body,
        grid=(N_STEP,),
        in_specs=[
            pl.BlockSpec((TAB_BLK, HIDDEN), lambda g: (jnp.minimum(g, N_ANS_STEP - 1), 0)),
            pl.BlockSpec((OCR_BLK, OCR_NUM, HIDDEN),
                         lambda g: (jnp.clip(g - N_ANS_STEP, 0, BATCH // OCR_BLK - 1), 0, 0)),
            pl.BlockSpec((4, HIDDEN), lambda g: (0, 0)),
        ],
        out_specs=pl.BlockSpec((TAB_BLK, HIDDEN), lambda g: (g, 0)),
        out_shape=jax.ShapeDtypeStruct((TAB_ROWS, HIDDEN), jnp.float32),
    )(ans_emb, ocr_emb, p)


def _emb_body(pe_ref, tt_ref, p_ref, o_ref):
    rid = lax.broadcasted_iota(jnp.int32, (2 * SEQ, 1), 0)
    x = pe_ref[...] + jnp.where(rid < SEQ, tt_ref[0], tt_ref[1])
    o_ref[...] = _ln(x, p_ref[0], p_ref[1])


def _emb_call(pe2, tt_table, p):
    return pl.pallas_call(
        _emb_body,
        grid=(1,),
        in_specs=[
            pl.BlockSpec((2 * SEQ, HIDDEN), lambda g: (0, 0)),
            pl.BlockSpec((2, HIDDEN), lambda g: (0, 0)),
            pl.BlockSpec((2, HIDDEN), lambda g: (0, 0)),
        ],
        out_specs=pl.BlockSpec((2 * SEQ, HIDDEN), lambda g: (0, 0)),
        out_shape=jax.ShapeDtypeStruct((2 * SEQ, HIDDEN), jnp.float32),
    )(pe2, tt_table, p)


def _sc_gather(table, emb_tab, labels):
    info = plsc.get_sparse_core_info()
    nc, ns = info.num_cores, info.num_subcores
    nw = nc * ns
    b_per_w = BATCH // nw
    mesh = plsc.VectorSubcoreMesh(core_axis_name="c", subcore_axis_name="s")

    chunk = 64
    n_chunk = b_per_w * SEQ // chunk

    @functools.partial(
        pl.kernel,
        mesh=mesh,
        out_type=jax.ShapeDtypeStruct((BATCH, SEQ, HIDDEN), jnp.float32),
        scratch_types=[
            pltpu.VMEM((b_per_w, SEQ), jnp.int32),
            pltpu.VMEM((3, chunk), jnp.int32),
            pltpu.VMEM((3, chunk), jnp.int32),
            pltpu.VMEM((3, 2 * chunk, HIDDEN), jnp.float32),
            pltpu.SemaphoreType.DMA,
            pltpu.SemaphoreType.DMA,
        ],
    )
    def k(table_hbm, emb_hbm, lbl_hbm, out_hbm, lbl_v, gi_v, ei_v, rows_v, sem_g, sem_s):
        wid = lax.axis_index("s") * nc + lax.axis_index("c")
        pltpu.sync_copy(lbl_hbm.at[pl.ds(wid * b_per_w, b_per_w)], lbl_v)

        def fire_gather(c):
            bi, half = c // 2, c % 2
            b = wid * b_per_w + bi
            slot = c % 3
            for j in range(chunk // LANES):
                v = lbl_v[bi, pl.ds(half * chunk + j * LANES, LANES)]
                is_ocr = v >= ANS_NUM
                gi_v[slot, pl.ds(j * LANES, LANES)] = jnp.where(
                    is_ocr, v + (OCR_BASE - ANS_NUM) + b * OCR_NUM, v)
                s_vec = lax.iota(jnp.int32, LANES) + (half * chunk + j * LANES)
                ei_v[slot, pl.ds(j * LANES, LANES)] = jnp.where(is_ocr, s_vec + SEQ, s_vec)
            g1 = pltpu.async_copy(table_hbm.at[gi_v.at[slot]],
                                  rows_v.at[slot, pl.ds(0, chunk)], sem_g)
            g2 = pltpu.async_copy(emb_hbm.at[ei_v.at[slot]],
                                  rows_v.at[slot, pl.ds(chunk, chunk)], sem_g)
            return g1, g2

        gathers = [None] * n_chunk
        scats = [None] * n_chunk
        gathers[0] = fire_gather(0)
        gathers[1] = fire_gather(1)
        for c in range(n_chunk):
            slot = c % 3
            if c + 2 < n_chunk:
                if c >= 1:
                    scats[c - 1].wait()
                gathers[c + 2] = fire_gather(c + 2)
            g1, g2 = gathers[c]
            g1.wait()
            g2.wait()

            def add_row(s, carry, slot=slot):
                for h in range(HIDDEN // LANES):
                    sl = pl.ds(h * LANES, LANES)
                    plsc.addupdate(rows_v.at[slot, s, sl], rows_v[slot, s + chunk, sl])
                return carry

            lax.fori_loop(0, chunk, add_row, 0)
            bi, half = c // 2, c % 2
            b = wid * b_per_w + bi
            scats[c] = pltpu.async_copy(rows_v.at[slot, pl.ds(0, chunk)],
                                        out_hbm.at[b, pl.ds(half * chunk, chunk)], sem_s)
        scats[n_chunk - 3].wait()
        scats[n_chunk - 2].wait()
        scats[n_chunk - 1].wait()

    return k(table, emb_tab, labels)


def kernel(ans_emb, ocr_emb, labels, tt_table, ans_ln_w, ans_ln_b, ocr_ln_w, ocr_ln_b, emb_ln_w, emb_ln_b):
    table = _tab_call(ans_emb, ocr_emb, jnp.stack([ans_ln_w, ans_ln_b, ocr_ln_w, ocr_ln_b]))
    emb_tab = _emb_call(jnp.asarray(_PE2), tt_table, jnp.stack([emb_ln_w, emb_ln_b]))
    return _sc_gather(table, emb_tab, labels.astype(jnp.int32))


# speedup vs baseline: 1.3625x; 1.0437x over previous
import functools
import math

import jax
import jax.numpy as jnp
import numpy as np
from jax import lax
from jax.experimental import pallas as pl
from jax.experimental.pallas import tpu as pltpu
from jax.experimental.pallas import tpu_sc as plsc

HIDDEN = 256
ANS_NUM = 5000
OCR_NUM = 100
BATCH = 128
SEQ = 128
LN_EPS = 1e-12
LANES = 16

OCR_BASE = 5600
EMB_BASE = OCR_BASE + BATCH * OCR_NUM
TAB_BLK = 800
TAB_ROWS = EMB_BASE + TAB_BLK
OCR_BLK = 8
N_ANS_STEP = OCR_BASE // TAB_BLK
N_EMB_STEP = EMB_BASE // TAB_BLK
N_STEP = TAB_ROWS // TAB_BLK


def _make_pe(d_model=HIDDEN, max_len=SEQ):
    position = np.arange(max_len, dtype=np.float64)[:, None]
    div_term = np.exp(np.arange(0, d_model, 2, dtype=np.float64) * (-math.log(10000.0) / d_model))
    pe = np.zeros((max_len, d_model), dtype=np.float32)
    pe[:, 0::2] = np.sin(position / div_term)
    pe[:, 1::2] = np.cos(position / div_term)
    return pe


_PE2PAD = np.concatenate(
    [_make_pe(), _make_pe(), np.zeros((TAB_BLK - 2 * SEQ, HIDDEN), np.float32)], axis=0
)


def _ln(x, w, b):
    n = x.shape[-1]
    mu = jnp.mean(x, axis=-1, keepdims=True)
    var = jnp.maximum(jnp.mean(x * x, axis=-1, keepdims=True) - mu * mu, 0.0)
    return (x - mu) * lax.rsqrt(var + LN_EPS) * w + b


def _tab_body(ans_ref, ocr_ref, pe_ref, tt_ref, p_ref, o_ref):
    g = pl.program_id(0)

    @pl.when(g < N_ANS_STEP)
    def _():
        o_ref[...] = _ln(ans_ref[...], p_ref[0], p_ref[1])

    @pl.when(jnp.logical_and(g >= N_ANS_STEP, g < N_EMB_STEP))
    def _():
        y = _ln(ocr_ref[...], p_ref[2], p_ref[3])
        for i in range(OCR_BLK):
            o_ref[pl.ds(i * OCR_NUM, OCR_NUM), :] = y[i]

    @pl.when(g >= N_EMB_STEP)
    def _():
        rid = lax.broadcasted_iota(jnp.int32, (TAB_BLK, 1), 0)
        x = pe_ref[...] + jnp.where(rid < SEQ, tt_ref[0], tt_ref[1])
        o_ref[...] = _ln(x, p_ref[4], p_ref[5])


def _tab_call(ans_emb, ocr_emb, pe2, tt_table, p):
    return pl.pallas_call(
        _tab_body,
        grid=(N_STEP,),
        in_specs=[
            pl.BlockSpec((TAB_BLK, HIDDEN), lambda g: (jnp.minimum(g, N_ANS_STEP - 1), 0)),
            pl.BlockSpec((OCR_BLK, OCR_NUM, HIDDEN),
                         lambda g: (jnp.clip(g - N_ANS_STEP, 0, BATCH // OCR_BLK - 1), 0, 0)),
            pl.BlockSpec((TAB_BLK, HIDDEN), lambda g: (0, 0)),
            pl.BlockSpec((2, HIDDEN), lambda g: (0, 0)),
            pl.BlockSpec((6, HIDDEN), lambda g: (0, 0)),
        ],
        out_specs=pl.BlockSpec((TAB_BLK, HIDDEN), lambda g: (g, 0)),
        out_shape=jax.ShapeDtypeStruct((TAB_ROWS, HIDDEN), jnp.float32),
    )(ans_emb, ocr_emb, pe2, tt_table, p)


def _sc_gather(table, labels):
    info = plsc.get_sparse_core_info()
    nc, ns = info.num_cores, info.num_subcores
    nw = nc * ns
    b_per_w = BATCH // nw
    mesh = plsc.VectorSubcoreMesh(core_axis_name="c", subcore_axis_name="s")

    chunk = 64
    n_chunk = b_per_w * SEQ // chunk

    @functools.partial(
        pl.kernel,
        mesh=mesh,
        out_type=jax.ShapeDtypeStruct((BATCH, SEQ, HIDDEN), jnp.float32),
        scratch_types=[
            pltpu.VMEM((b_per_w, SEQ), jnp.int32),
            pltpu.VMEM((3, chunk), jnp.int32),
            pltpu.VMEM((3, chunk), jnp.int32),
            pltpu.VMEM((3, 2 * chunk, HIDDEN), jnp.float32),
            pltpu.SemaphoreType.DMA,
            pltpu.SemaphoreType.DMA,
        ],
    )
    def k(table_hbm, lbl_hbm, out_hbm, lbl_v, gi_v, ei_v, rows_v, sem_g, sem_s):
        wid = lax.axis_index("s") * nc + lax.axis_index("c")
        pltpu.sync_copy(lbl_hbm.at[pl.ds(wid * b_per_w, b_per_w)], lbl_v)

        def fire_gather(c):
            bi, half = c // 2, c % 2
            b = wid * b_per_w + bi
            slot = c % 3
            for j in range(chunk // LANES):
                v = lbl_v[bi, pl.ds(half * chunk + j * LANES, LANES)]
                is_ocr = v >= ANS_NUM
                gi_v[slot, pl.ds(j * LANES, LANES)] = jnp.where(
                    is_ocr, v + (OCR_BASE - ANS_NUM) + b * OCR_NUM, v)
                s_vec = lax.iota(jnp.int32, LANES) + (half * chunk + j * LANES)
                ei_v[slot, pl.ds(j * LANES, LANES)] = jnp.where(
                    is_ocr, s_vec + (EMB_BASE + SEQ), s_vec + EMB_BASE)
            g1 = pltpu.async_copy(table_hbm.at[gi_v.at[slot]],
                                  rows_v.at[slot, pl.ds(0, chunk)], sem_g)
            g2 = pltpu.async_copy(table_hbm.at[ei_v.at[slot]],
                                  rows_v.at[slot, pl.ds(chunk, chunk)], sem_g)
            return g1, g2

        gathers = [None] * n_chunk
        scats = [None] * n_chunk
        gathers[0] = fire_gather(0)
        gathers[1] = fire_gather(1)
        for c in range(n_chunk):
            slot = c % 3
            if c + 2 < n_chunk:
                if c >= 1:
                    scats[c - 1].wait()
                gathers[c + 2] = fire_gather(c + 2)
            g1, g2 = gathers[c]
            g1.wait()
            g2.wait()

            def add_row(s, carry, slot=slot):
                for h in range(HIDDEN // LANES):
                    sl = pl.ds(h * LANES, LANES)
                    plsc.addupdate(rows_v.at[slot, s, sl], rows_v[slot, s + chunk, sl])
                return carry

            lax.fori_loop(0, chunk, add_row, 0)
            bi, half = c // 2, c % 2
            b = wid * b_per_w + bi
            scats[c] = pltpu.async_copy(rows_v.at[slot, pl.ds(0, chunk)],
                                        out_hbm.at[b, pl.ds(half * chunk, chunk)], sem_s)
        scats[n_chunk - 3].wait()
        scats[n_chunk - 2].wait()
        scats[n_chunk - 1].wait()

    return k(table, labels)


def kernel(ans_emb, ocr_emb, labels, tt_table, ans_ln_w, ans_ln_b, ocr_ln_w, ocr_ln_b, emb_ln_w, emb_ln_b):
    p = jnp.stack([ans_ln_w, ans_ln_b, ocr_ln_w, ocr_ln_b, emb_ln_w, emb_ln_b])
    table = _tab_call(ans_emb, ocr_emb, jnp.asarray(_PE2PAD), tt_table, p)
    return _sc_gather(table, labels.astype(jnp.int32))
